# Initial kernel scaffold; baseline (speedup 1.0000x reference)
import jax
import jax.numpy as jnp
from jax.experimental import pallas as pl


def kernel(x, edge_index, W_l1, b_l1, W_r1, gamma1, beta1, W_l2, b_l2, W_r2):
    raise NotImplementedError("write your pallas kernel here")


# speedup vs baseline: 3.9707x; 3.9707x over previous
import functools

import jax
import jax.numpy as jnp
from jax import lax
from jax.experimental import pallas as pl
from jax.experimental.pallas import tpu as pltpu
from jax.experimental.pallas import tpu_sc as plsc

NC = 2
NS = 16
NW = NC * NS
CH = 80

_F32 = jnp.float32


def _sc_mesh():
    return plsc.VectorSubcoreMesh(
        core_axis_name="c", subcore_axis_name="s", num_cores=NC, num_subcores=NS
    )


def _round_up(v, m):
    return (v + m - 1) // m * m


def _copy_out_slices(N, rz):
    slices = []
    for s in range(NS):
        r0 = s * rz
        slices.append((r0, min(rz, N - r0) if r0 < N else 0))
    return slices


@functools.cache
def _build_agg1(N, E, D):
    epw = E // NW
    nch = epw // CH
    NP = _round_up(N, 8 * NS)
    rz = NP // NS

    @functools.partial(
        pl.kernel,
        out_type=(
            jax.ShapeDtypeStruct((NC, N, D), _F32),
            jax.ShapeDtypeStruct((NC, N, D), _F32),
        ),
        mesh=_sc_mesh(),
        scratch_types=[
            pltpu.VMEM_SHARED((NP, D), _F32),
            pltpu.VMEM((CH,), jnp.int32),
            pltpu.VMEM((CH,), jnp.int32),
            pltpu.VMEM((CH, D), _F32),
            pltpu.SemaphoreType.DMA,
        ],
    )
    def agg1(x_hbm, src_hbm, dst_hbm, znf_hbm, ones_hbm,
             sum_out, cnt_out, acc, src_v, dst_v, rows_v, sem):
        c = lax.axis_index("c")
        s = lax.axis_index("s")
        wid = s * NC + c
        z0 = pl.multiple_of(s * rz, 8)
        pltpu.sync_copy(znf_hbm, acc.at[pl.ds(z0, rz)])
        plsc.subcore_barrier()
        base = wid * epw

        def step(i, carry):
            e0 = pl.multiple_of(base + i * CH, 8)
            pltpu.sync_copy(src_hbm.at[pl.ds(e0, CH)], src_v)
            pltpu.sync_copy(dst_hbm.at[pl.ds(e0, CH)], dst_v)
            pltpu.async_copy(x_hbm.at[src_v], rows_v, sem).wait()
            pltpu.sync_copy(rows_v, acc.at[dst_v], add=True)
            return carry

        lax.fori_loop(0, nch, step, 0)
        plsc.subcore_barrier()
        for r0, nr in _copy_out_slices(N, rz):
            @pl.when(s == r0 // rz)
            def _():
                pltpu.sync_copy(acc.at[pl.ds(r0, nr)],
                                sum_out.at[c, pl.ds(r0, nr)])
        plsc.subcore_barrier()
        pltpu.sync_copy(znf_hbm, acc.at[pl.ds(z0, rz)])
        pltpu.sync_copy(ones_hbm, rows_v)
        plsc.subcore_barrier()

        def cstep(i, carry):
            e0 = pl.multiple_of(base + i * CH, 8)
            pltpu.sync_copy(dst_hbm.at[pl.ds(e0, CH)], dst_v)
            pltpu.sync_copy(rows_v, acc.at[dst_v], add=True)
            return carry

        lax.fori_loop(0, nch, cstep, 0)
        plsc.subcore_barrier()
        for r0, nr in _copy_out_slices(N, rz):
            @pl.when(s == r0 // rz)
            def _():
                pltpu.sync_copy(acc.at[pl.ds(r0, nr)],
                                cnt_out.at[c, pl.ds(r0, nr)])

    return agg1


@functools.cache
def _build_agg2(N, E, D):
    ept = E // NS
    nch = ept // CH
    NP = _round_up(N, 8 * NS)
    rz = NP // NS

    @functools.partial(
        pl.kernel,
        out_type=jax.ShapeDtypeStruct((NC, N, D), _F32),
        mesh=_sc_mesh(),
        scratch_types=[
            pltpu.VMEM_SHARED((NP, D), _F32),
            pltpu.VMEM((CH,), jnp.int32),
            pltpu.VMEM((CH,), jnp.int32),
            pltpu.VMEM((CH, D), _F32),
            pltpu.SemaphoreType.DMA,
        ],
    )
    def agg2(hcat_hbm, src2_hbm, dst_hbm, znf_hbm,
             sum_out, acc, src_v, dst_v, rows_v, sem):
        c = lax.axis_index("c")
        s = lax.axis_index("s")
        z0 = pl.multiple_of(s * rz, 8)
        pltpu.sync_copy(znf_hbm, acc.at[pl.ds(z0, rz)])
        plsc.subcore_barrier()
        sbase = c * E + s * ept
        dbase = s * ept

        def step(i, carry):
            s0 = pl.multiple_of(sbase + i * CH, 8)
            d0 = pl.multiple_of(dbase + i * CH, 8)
            pltpu.sync_copy(src2_hbm.at[pl.ds(s0, CH)], src_v)
            pltpu.sync_copy(dst_hbm.at[pl.ds(d0, CH)], dst_v)
            pltpu.async_copy(hcat_hbm.at[src_v], rows_v, sem).wait()
            pltpu.sync_copy(rows_v, acc.at[dst_v], add=True)
            return carry

        lax.fori_loop(0, nch, step, 0)
        plsc.subcore_barrier()
        for r0, nr in _copy_out_slices(N, rz):
            @pl.when(s == r0 // rz)
            def _():
                pltpu.sync_copy(acc.at[pl.ds(r0, nr)],
                                sum_out.at[c, pl.ds(r0, nr)])

    return agg2


def _tc_b_body(nb, sum1_ref, cntp_ref, x_ref, wl_ref, bl_ref, wr_ref,
               hpre_ref, stats_ref, rcp_ref, acc_ref):
    i = pl.program_id(0)
    cnt = cntp_ref[0, :, 0:1] + cntp_ref[1, :, 0:1]
    rcp = 1.0 / jnp.clip(cnt, 1.0, None)
    rcp_ref[...] = jnp.broadcast_to(rcp, rcp_ref.shape)
    mean1 = (sum1_ref[0] + sum1_ref[1]) * rcp
    hp = (jnp.dot(mean1, wl_ref[...], preferred_element_type=_F32)
          + jnp.dot(x_ref[...], wr_ref[...], preferred_element_type=_F32)
          + bl_ref[...])
    hpre_ref[...] = hp

    @pl.when(i == 0)
    def _():
        acc_ref[...] = jnp.zeros_like(acc_ref)

    acc_ref[0:1, :] += jnp.sum(hp, axis=0, keepdims=True)
    acc_ref[1:2, :] += jnp.sum(hp * hp, axis=0, keepdims=True)

    @pl.when(i == nb - 1)
    def _():
        stats_ref[...] = acc_ref[...]


def _tc_b2_body(N, hpre_ref, stats_ref, g_ref, b_ref, hcat_ref):
    inv_n = 1.0 / N
    mu = stats_ref[0:1, :] * inv_n
    var = stats_ref[1:2, :] * inv_n - mu * mu
    scale = g_ref[...] * lax.rsqrt(var + 1e-5)
    shift = b_ref[...] - mu * scale
    h = hpre_ref[...] * scale + shift
    h = jnp.where(h >= 0, h, 0.01 * h)
    d = h.shape[1] // 2
    hcat_ref[0] = h[:, :d]
    hcat_ref[1] = h[:, d:]


def _tc_d_body(sum2_ref, rcp_ref, hcat_ref, wl_ref, bl_ref, wr_ref, out_ref):
    rcp = rcp_ref[:, 0:1]
    d = wl_ref.shape[0] // 2
    out_ref[...] = (
        jnp.dot(sum2_ref[0] * rcp, wl_ref[:d], preferred_element_type=_F32)
        + jnp.dot(sum2_ref[1] * rcp, wl_ref[d:], preferred_element_type=_F32)
        + jnp.dot(hcat_ref[0], wr_ref[:d], preferred_element_type=_F32)
        + jnp.dot(hcat_ref[1], wr_ref[d:], preferred_element_type=_F32)
        + bl_ref[...]
    )


def kernel(x, edge_index, W_l1, b_l1, W_r1, gamma1, beta1, W_l2, b_l2, W_r2):
    N, d_in = x.shape
    E = edge_index.shape[1]
    d_h = W_l1.shape[0]
    d_out = W_l2.shape[0]
    dh2 = d_h // 2
    NP = _round_up(N, 8 * NS)
    rz = NP // NS

    src = edge_index[0]
    dst = edge_index[1]
    src2 = jnp.concatenate([src, src + N])
    znf = jnp.zeros((rz, d_in), _F32)
    znh = jnp.zeros((rz, dh2), _F32)
    ones = jnp.ones((CH, d_in), _F32)

    sum1, cntp = _build_agg1(N, E, d_in)(x, src, dst, znf, ones)

    nb = 10
    R = N // nb
    hpre, stats, rcp16 = pl.pallas_call(
        functools.partial(_tc_b_body, nb),
        grid=(nb,),
        in_specs=[
            pl.BlockSpec((NC, R, d_in), lambda i: (0, i, 0)),
            pl.BlockSpec((NC, R, d_in), lambda i: (0, i, 0)),
            pl.BlockSpec((R, d_in), lambda i: (i, 0)),
            pl.BlockSpec((d_in, d_h), lambda i: (0, 0)),
            pl.BlockSpec((1, d_h), lambda i: (0, 0)),
            pl.BlockSpec((d_in, d_h), lambda i: (0, 0)),
        ],
        out_specs=[
            pl.BlockSpec((R, d_h), lambda i: (i, 0)),
            pl.BlockSpec((8, d_h), lambda i: (0, 0)),
            pl.BlockSpec((R, 16), lambda i: (i, 0)),
        ],
        out_shape=[
            jax.ShapeDtypeStruct((N, d_h), _F32),
            jax.ShapeDtypeStruct((8, d_h), _F32),
            jax.ShapeDtypeStruct((N, 16), _F32),
        ],
        scratch_shapes=[pltpu.VMEM((8, d_h), _F32)],
    )(sum1, cntp, x, W_l1.T, b_l1.reshape(1, -1), W_r1.T)

    hcat = pl.pallas_call(
        functools.partial(_tc_b2_body, N),
        grid=(nb,),
        in_specs=[
            pl.BlockSpec((R, d_h), lambda i: (i, 0)),
            pl.BlockSpec((8, d_h), lambda i: (0, 0)),
            pl.BlockSpec((1, d_h), lambda i: (0, 0)),
            pl.BlockSpec((1, d_h), lambda i: (0, 0)),
        ],
        out_specs=pl.BlockSpec((NC, R, dh2), lambda i: (0, i, 0)),
        out_shape=jax.ShapeDtypeStruct((NC, N, dh2), _F32),
    )(hpre, stats, gamma1.reshape(1, -1), beta1.reshape(1, -1))

    sum2 = _build_agg2(N, E, dh2)(hcat.reshape(NC * N, dh2), src2, dst, znh)

    node_emb = pl.pallas_call(
        _tc_d_body,
        grid=(nb,),
        in_specs=[
            pl.BlockSpec((NC, R, dh2), lambda i: (0, i, 0)),
            pl.BlockSpec((R, 16), lambda i: (i, 0)),
            pl.BlockSpec((NC, R, dh2), lambda i: (0, i, 0)),
            pl.BlockSpec((d_h, d_out), lambda i: (0, 0)),
            pl.BlockSpec((1, d_out), lambda i: (0, 0)),
            pl.BlockSpec((d_h, d_out), lambda i: (0, 0)),
        ],
        out_specs=pl.BlockSpec((R, d_out), lambda i: (i, 0)),
        out_shape=jax.ShapeDtypeStruct((N, d_out), _F32),
    )(sum2, rcp16, hcat, W_l2.T, b_l2.reshape(1, -1), W_r2.T)

    return node_emb


# speedup vs baseline: 7.0355x; 1.7719x over previous
import functools

import jax
import jax.numpy as jnp
from jax import lax
from jax.experimental import pallas as pl
from jax.experimental.pallas import tpu as pltpu
from jax.experimental.pallas import tpu_sc as plsc

NC = 2
NS = 16
NW = NC * NS
CH = 80

_F32 = jnp.float32


def _sc_mesh():
    return plsc.VectorSubcoreMesh(
        core_axis_name="c", subcore_axis_name="s", num_cores=NC, num_subcores=NS
    )


def _round_up(v, m):
    return (v + m - 1) // m * m


def _copy_out_slices(N, rz):
    slices = []
    for s in range(NS):
        r0 = s * rz
        slices.append((r0, min(rz, N - r0) if r0 < N else 0))
    return slices


def _pipelined_pass(table_hbm, acc, src_blk, dst_blk, rows_a, rows_b, sem, pb):
    pltpu.async_copy(table_hbm.at[src_blk.at[0]], rows_a, sem)

    def step(i, carry):
        pltpu.make_async_copy(table_hbm.at[pl.ds(0, CH)], rows_a, sem).wait()

        @pl.when(i + 1 < pb)
        def _():
            @pl.when(lax.rem(i, 2) == 1)
            def _():
                pltpu.async_copy(table_hbm.at[src_blk.at[i + 1]], rows_a, sem)
            @pl.when(lax.rem(i, 2) == 0)
            def _():
                pltpu.async_copy(table_hbm.at[src_blk.at[i + 1]], rows_b, sem)

        @pl.when(lax.rem(i, 2) == 0)
        def _():
            pltpu.sync_copy(rows_a, acc.at[dst_blk.at[i]], add=True)
        @pl.when(lax.rem(i, 2) == 1)
        def _():
            pltpu.sync_copy(rows_b, acc.at[dst_blk.at[i]], add=True)
        return carry

    lax.fori_loop(0, pb, step, 0)


PB = 25


@functools.cache
def _build_agg1(N, E, D):
    epw = E // NW
    nch = epw // CH
    NP = _round_up(N, 8 * NS)
    rz = NP // NS

    @functools.partial(
        pl.kernel,
        out_type=(
            jax.ShapeDtypeStruct((NC, N, D), _F32),
            jax.ShapeDtypeStruct((NC, N, D), _F32),
        ),
        mesh=_sc_mesh(),
        scratch_types=[
            pltpu.VMEM_SHARED((NP, D), _F32),
            pltpu.VMEM((PB, CH), jnp.int32),
            pltpu.VMEM((PB, CH), jnp.int32),
            pltpu.VMEM((CH, D), _F32),
            pltpu.VMEM((CH, D), _F32),
            pltpu.SemaphoreType.DMA,
            pltpu.SemaphoreType.DMA,
        ],
    )
    def agg1(x_hbm, src3_hbm, dst3_hbm, znf_hbm, ones_hbm,
             sum_out, cnt_out, acc, src_blk, dst_blk, rows_a, rows_b,
             sem, sem2):
        c = lax.axis_index("c")
        s = lax.axis_index("s")
        wid = s * NC + c
        z0 = pl.multiple_of(s * rz, 8)
        pltpu.sync_copy(znf_hbm, acc.at[pl.ds(z0, rz)])
        plsc.subcore_barrier()

        def ppass(p, carry):
            pltpu.sync_copy(src3_hbm.at[wid, p], src_blk)
            pltpu.sync_copy(dst3_hbm.at[wid, p], dst_blk)
            _pipelined_pass(x_hbm, acc, src_blk, dst_blk, rows_a, rows_b,
                            sem, PB)
            return carry

        lax.fori_loop(0, nch // PB, ppass, 0)
        plsc.subcore_barrier()
        for r0, nr in _copy_out_slices(N, rz):
            @pl.when(s == r0 // rz)
            def _():
                pltpu.sync_copy(acc.at[pl.ds(r0, nr)],
                                sum_out.at[c, pl.ds(r0, nr)])
        plsc.subcore_barrier()
        pltpu.sync_copy(znf_hbm, acc.at[pl.ds(z0, rz)])
        pltpu.sync_copy(ones_hbm, rows_a)
        plsc.subcore_barrier()

        def cpass(p, carry):
            pltpu.sync_copy(dst3_hbm.at[wid, p], dst_blk)
            pltpu.async_copy(rows_a, acc.at[dst_blk.at[0]], sem2, add=True)

            def cstep(i, carry2):
                pltpu.async_copy(rows_a, acc.at[dst_blk.at[i + 1]], sem2,
                                 add=True)
                pltpu.make_async_copy(rows_a, acc.at[dst_blk.at[0]],
                                      sem2).wait()
                return carry2

            lax.fori_loop(0, PB - 1, cstep, 0)
            pltpu.make_async_copy(rows_a, acc.at[dst_blk.at[0]], sem2).wait()
            return carry

        lax.fori_loop(0, nch // PB, cpass, 0)
        plsc.subcore_barrier()
        for r0, nr in _copy_out_slices(N, rz):
            @pl.when(s == r0 // rz)
            def _():
                pltpu.sync_copy(acc.at[pl.ds(r0, nr)],
                                cnt_out.at[c, pl.ds(r0, nr)])

    return agg1


@functools.cache
def _build_agg2(N, E, D):
    ept = E // NS
    nch = ept // CH
    NP = _round_up(N, 8 * NS)
    rz = NP // NS

    @functools.partial(
        pl.kernel,
        out_type=jax.ShapeDtypeStruct((NC, N, D), _F32),
        mesh=_sc_mesh(),
        scratch_types=[
            pltpu.VMEM_SHARED((NP, D), _F32),
            pltpu.VMEM((PB, CH), jnp.int32),
            pltpu.VMEM((PB, CH), jnp.int32),
            pltpu.VMEM((CH, D), _F32),
            pltpu.VMEM((CH, D), _F32),
            pltpu.SemaphoreType.DMA,
        ],
    )
    def agg2(hcat_hbm, src4_hbm, dst3_hbm, znf_hbm,
             sum_out, acc, src_blk, dst_blk, rows_a, rows_b, sem):
        c = lax.axis_index("c")
        s = lax.axis_index("s")
        z0 = pl.multiple_of(s * rz, 8)
        pltpu.sync_copy(znf_hbm, acc.at[pl.ds(z0, rz)])
        plsc.subcore_barrier()

        def ppass(p, carry):
            pltpu.sync_copy(src4_hbm.at[c, s, p], src_blk)
            pltpu.sync_copy(dst3_hbm.at[s, p], dst_blk)
            _pipelined_pass(hcat_hbm, acc, src_blk, dst_blk, rows_a, rows_b,
                            sem, PB)
            return carry

        lax.fori_loop(0, nch // PB, ppass, 0)
        plsc.subcore_barrier()
        for r0, nr in _copy_out_slices(N, rz):
            @pl.when(s == r0 // rz)
            def _():
                pltpu.sync_copy(acc.at[pl.ds(r0, nr)],
                                sum_out.at[c, pl.ds(r0, nr)])

    return agg2


def _tc_b_body(nb, sum1_ref, cntp_ref, x_ref, wl_ref, bl_ref, wr_ref,
               hpre_ref, stats_ref, rcp_ref, acc_ref):
    i = pl.program_id(0)
    cnt = cntp_ref[0, :, 0:1] + cntp_ref[1, :, 0:1]
    rcp = 1.0 / jnp.clip(cnt, 1.0, None)
    rcp_ref[...] = jnp.broadcast_to(rcp, rcp_ref.shape)
    mean1 = (sum1_ref[0] + sum1_ref[1]) * rcp
    hp = (jnp.dot(mean1, wl_ref[...], preferred_element_type=_F32)
          + jnp.dot(x_ref[...], wr_ref[...], preferred_element_type=_F32)
          + bl_ref[...])
    hpre_ref[...] = hp

    @pl.when(i == 0)
    def _():
        acc_ref[...] = jnp.zeros_like(acc_ref)

    acc_ref[0:1, :] += jnp.sum(hp, axis=0, keepdims=True)
    acc_ref[1:2, :] += jnp.sum(hp * hp, axis=0, keepdims=True)

    @pl.when(i == nb - 1)
    def _():
        stats_ref[...] = acc_ref[...]


def _tc_b2_body(N, hpre_ref, stats_ref, g_ref, b_ref, hcat_ref):
    inv_n = 1.0 / N
    mu = stats_ref[0:1, :] * inv_n
    var = stats_ref[1:2, :] * inv_n - mu * mu
    scale = g_ref[...] * lax.rsqrt(var + 1e-5)
    shift = b_ref[...] - mu * scale
    h = hpre_ref[...] * scale + shift
    h = jnp.where(h >= 0, h, 0.01 * h)
    d = h.shape[1] // 2
    hcat_ref[0] = h[:, :d]
    hcat_ref[1] = h[:, d:]


def _tc_d_body(sum2_ref, rcp_ref, hcat_ref, wl_ref, bl_ref, wr_ref, out_ref):
    rcp = rcp_ref[:, 0:1]
    d = wl_ref.shape[0] // 2
    out_ref[...] = (
        jnp.dot(sum2_ref[0] * rcp, wl_ref[:d], preferred_element_type=_F32)
        + jnp.dot(sum2_ref[1] * rcp, wl_ref[d:], preferred_element_type=_F32)
        + jnp.dot(hcat_ref[0], wr_ref[:d], preferred_element_type=_F32)
        + jnp.dot(hcat_ref[1], wr_ref[d:], preferred_element_type=_F32)
        + bl_ref[...]
    )


def kernel(x, edge_index, W_l1, b_l1, W_r1, gamma1, beta1, W_l2, b_l2, W_r2):
    N, d_in = x.shape
    E = edge_index.shape[1]
    d_h = W_l1.shape[0]
    d_out = W_l2.shape[0]
    dh2 = d_h // 2
    NP = _round_up(N, 8 * NS)
    rz = NP // NS

    src = edge_index[0]
    dst = edge_index[1]
    np1 = E // NW // CH // PB
    np2 = E // NS // CH // PB
    src3 = src.reshape(NW, np1, PB, CH)
    dst3 = dst.reshape(NW, np1, PB, CH)
    src4 = jnp.concatenate([src, src + N]).reshape(NC, NS, np2, PB, CH)
    dst3b = dst.reshape(NS, np2, PB, CH)
    znf = jnp.zeros((rz, d_in), _F32)
    znh = jnp.zeros((rz, dh2), _F32)
    ones = jnp.ones((CH, d_in), _F32)

    sum1, cntp = _build_agg1(N, E, d_in)(x, src3, dst3, znf, ones)

    nb = 10
    R = N // nb
    hpre, stats, rcp16 = pl.pallas_call(
        functools.partial(_tc_b_body, nb),
        grid=(nb,),
        in_specs=[
            pl.BlockSpec((NC, R, d_in), lambda i: (0, i, 0)),
            pl.BlockSpec((NC, R, d_in), lambda i: (0, i, 0)),
            pl.BlockSpec((R, d_in), lambda i: (i, 0)),
            pl.BlockSpec((d_in, d_h), lambda i: (0, 0)),
            pl.BlockSpec((1, d_h), lambda i: (0, 0)),
            pl.BlockSpec((d_in, d_h), lambda i: (0, 0)),
        ],
        out_specs=[
            pl.BlockSpec((R, d_h), lambda i: (i, 0)),
            pl.BlockSpec((8, d_h), lambda i: (0, 0)),
            pl.BlockSpec((R, 16), lambda i: (i, 0)),
        ],
        out_shape=[
            jax.ShapeDtypeStruct((N, d_h), _F32),
            jax.ShapeDtypeStruct((8, d_h), _F32),
            jax.ShapeDtypeStruct((N, 16), _F32),
        ],
        scratch_shapes=[pltpu.VMEM((8, d_h), _F32)],
    )(sum1, cntp, x, W_l1.T, b_l1.reshape(1, -1), W_r1.T)

    hcat = pl.pallas_call(
        functools.partial(_tc_b2_body, N),
        grid=(nb,),
        in_specs=[
            pl.BlockSpec((R, d_h), lambda i: (i, 0)),
            pl.BlockSpec((8, d_h), lambda i: (0, 0)),
            pl.BlockSpec((1, d_h), lambda i: (0, 0)),
            pl.BlockSpec((1, d_h), lambda i: (0, 0)),
        ],
        out_specs=pl.BlockSpec((NC, R, dh2), lambda i: (0, i, 0)),
        out_shape=jax.ShapeDtypeStruct((NC, N, dh2), _F32),
    )(hpre, stats, gamma1.reshape(1, -1), beta1.reshape(1, -1))

    sum2 = _build_agg2(N, E, dh2)(hcat.reshape(NC * N, dh2), src4, dst3b, znh)

    node_emb = pl.pallas_call(
        _tc_d_body,
        grid=(nb,),
        in_specs=[
            pl.BlockSpec((NC, R, dh2), lambda i: (0, i, 0)),
            pl.BlockSpec((R, 16), lambda i: (i, 0)),
            pl.BlockSpec((NC, R, dh2), lambda i: (0, i, 0)),
            pl.BlockSpec((d_h, d_out), lambda i: (0, 0)),
            pl.BlockSpec((1, d_out), lambda i: (0, 0)),
            pl.BlockSpec((d_h, d_out), lambda i: (0, 0)),
        ],
        out_specs=pl.BlockSpec((R, d_out), lambda i: (i, 0)),
        out_shape=jax.ShapeDtypeStruct((N, d_out), _F32),
    )(sum2, rcp16, hcat, W_l2.T, b_l2.reshape(1, -1), W_r2.T)

    return node_emb


# speedup vs baseline: 9.5245x; 1.3538x over previous
import functools

import jax
import jax.numpy as jnp
from jax import lax
from jax.experimental import pallas as pl
from jax.experimental.pallas import tpu as pltpu
from jax.experimental.pallas import tpu_sc as plsc

NC = 2
NS = 16
NW = NC * NS
CH = 80

_F32 = jnp.float32


def _sc_mesh():
    return plsc.VectorSubcoreMesh(
        core_axis_name="c", subcore_axis_name="s", num_cores=NC, num_subcores=NS
    )


def _round_up(v, m):
    return (v + m - 1) // m * m


def _copy_out_slices(N, rz):
    slices = []
    for s in range(NS):
        r0 = s * rz
        slices.append((r0, min(rz, N - r0) if r0 < N else 0))
    return slices


def _pipelined_pass(table_hbm, acc, src_blk, dst_blk, rows, sem, sem2, pb):
    pltpu.async_copy(table_hbm.at[src_blk.at[0]], rows[0], sem)
    pltpu.async_copy(table_hbm.at[src_blk.at[1]], rows[1], sem)

    def step(i, carry):
        pltpu.make_async_copy(table_hbm.at[pl.ds(0, CH)], rows[0], sem).wait()

        @pl.when(i > 0)
        def _():
            pltpu.make_async_copy(rows[0], acc.at[dst_blk.at[0]], sem2).wait()

        @pl.when(i + 2 < pb)
        def _():
            for r in range(3):
                @pl.when(lax.rem(i, 3) == r)
                def _():
                    pltpu.async_copy(table_hbm.at[src_blk.at[i + 2]],
                                     rows[(r + 2) % 3], sem)

        for r in range(3):
            @pl.when(lax.rem(i, 3) == r)
            def _():
                pltpu.async_copy(rows[r], acc.at[dst_blk.at[i]], sem2,
                                 add=True)
        return carry

    lax.fori_loop(0, pb, step, 0)
    pltpu.make_async_copy(rows[0], acc.at[dst_blk.at[0]], sem2).wait()


PB = 25


@functools.cache
def _build_agg1(N, E, D):
    epw = E // NW
    nch = epw // CH
    NP = _round_up(N, 8 * NS)
    rz = NP // NS

    @functools.partial(
        pl.kernel,
        out_type=(
            jax.ShapeDtypeStruct((NC, N, D), _F32),
            jax.ShapeDtypeStruct((NC, N, D), _F32),
        ),
        mesh=_sc_mesh(),
        scratch_types=[
            pltpu.VMEM_SHARED((NP, D), _F32),
            pltpu.VMEM((PB, CH), jnp.int32),
            pltpu.VMEM((PB, CH), jnp.int32),
            pltpu.VMEM((CH, D), _F32),
            pltpu.VMEM((CH, D), _F32),
            pltpu.VMEM((CH, D), _F32),
            pltpu.SemaphoreType.DMA,
            pltpu.SemaphoreType.DMA,
        ],
    )
    def agg1(x_hbm, src3_hbm, dst3_hbm, znf_hbm, ones_hbm,
             sum_out, cnt_out, acc, src_blk, dst_blk, rows_a, rows_b, rows_c,
             sem, sem2):
        c = lax.axis_index("c")
        s = lax.axis_index("s")
        wid = s * NC + c
        z0 = pl.multiple_of(s * rz, 8)
        pltpu.sync_copy(znf_hbm, acc.at[pl.ds(z0, rz)])
        plsc.subcore_barrier()

        def ppass(p, carry):
            pltpu.sync_copy(src3_hbm.at[wid, p], src_blk)
            pltpu.sync_copy(dst3_hbm.at[wid, p], dst_blk)
            _pipelined_pass(x_hbm, acc, src_blk, dst_blk,
                            (rows_a, rows_b, rows_c), sem, sem2, PB)
            return carry

        lax.fori_loop(0, nch // PB, ppass, 0)
        plsc.subcore_barrier()
        for r0, nr in _copy_out_slices(N, rz):
            @pl.when(s == r0 // rz)
            def _():
                pltpu.sync_copy(acc.at[pl.ds(r0, nr)],
                                sum_out.at[c, pl.ds(r0, nr)])
        plsc.subcore_barrier()
        pltpu.sync_copy(znf_hbm, acc.at[pl.ds(z0, rz)])
        pltpu.sync_copy(ones_hbm, rows_a)
        plsc.subcore_barrier()

        def cpass(p, carry):
            pltpu.sync_copy(dst3_hbm.at[wid, p], dst_blk)
            pltpu.async_copy(rows_a, acc.at[dst_blk.at[0]], sem2, add=True)

            def cstep(i, carry2):
                pltpu.async_copy(rows_a, acc.at[dst_blk.at[i + 1]], sem2,
                                 add=True)
                pltpu.make_async_copy(rows_a, acc.at[dst_blk.at[0]],
                                      sem2).wait()
                return carry2

            lax.fori_loop(0, PB - 1, cstep, 0)
            pltpu.make_async_copy(rows_a, acc.at[dst_blk.at[0]], sem2).wait()
            return carry

        lax.fori_loop(0, nch // PB, cpass, 0)
        plsc.subcore_barrier()
        for r0, nr in _copy_out_slices(N, rz):
            @pl.when(s == r0 // rz)
            def _():
                pltpu.sync_copy(acc.at[pl.ds(r0, nr)],
                                cnt_out.at[c, pl.ds(r0, nr)])

    return agg1


@functools.cache
def _build_agg2(N, E, D):
    ept = E // NS
    nch = ept // CH
    NP = _round_up(N, 8 * NS)
    rz = NP // NS

    @functools.partial(
        pl.kernel,
        out_type=jax.ShapeDtypeStruct((NC, N, D), _F32),
        mesh=_sc_mesh(),
        scratch_types=[
            pltpu.VMEM_SHARED((NP, D), _F32),
            pltpu.VMEM((PB, CH), jnp.int32),
            pltpu.VMEM((PB, CH), jnp.int32),
            pltpu.VMEM((CH, D), _F32),
            pltpu.VMEM((CH, D), _F32),
            pltpu.VMEM((CH, D), _F32),
            pltpu.SemaphoreType.DMA,
            pltpu.SemaphoreType.DMA,
        ],
    )
    def agg2(hcat_hbm, src4_hbm, dst3_hbm, znf_hbm,
             sum_out, acc, src_blk, dst_blk, rows_a, rows_b, rows_c,
             sem, sem2):
        c = lax.axis_index("c")
        s = lax.axis_index("s")
        z0 = pl.multiple_of(s * rz, 8)
        pltpu.sync_copy(znf_hbm, acc.at[pl.ds(z0, rz)])
        plsc.subcore_barrier()

        def ppass(p, carry):
            pltpu.sync_copy(src4_hbm.at[c, s, p], src_blk)
            pltpu.sync_copy(dst3_hbm.at[s, p], dst_blk)
            _pipelined_pass(hcat_hbm, acc, src_blk, dst_blk,
                            (rows_a, rows_b, rows_c), sem, sem2, PB)
            return carry

        lax.fori_loop(0, nch // PB, ppass, 0)
        plsc.subcore_barrier()
        for r0, nr in _copy_out_slices(N, rz):
            @pl.when(s == r0 // rz)
            def _():
                pltpu.sync_copy(acc.at[pl.ds(r0, nr)],
                                sum_out.at[c, pl.ds(r0, nr)])

    return agg2


def _tc_b_body(nb, sum1_ref, cntp_ref, x_ref, wl_ref, bl_ref, wr_ref,
               hpre_ref, stats_ref, rcp_ref, acc_ref):
    i = pl.program_id(0)
    cnt = cntp_ref[0, :, 0:1] + cntp_ref[1, :, 0:1]
    rcp = 1.0 / jnp.clip(cnt, 1.0, None)
    rcp_ref[...] = jnp.broadcast_to(rcp, rcp_ref.shape)
    mean1 = (sum1_ref[0] + sum1_ref[1]) * rcp
    hp = (jnp.dot(mean1, wl_ref[...], preferred_element_type=_F32)
          + jnp.dot(x_ref[...], wr_ref[...], preferred_element_type=_F32)
          + bl_ref[...])
    hpre_ref[...] = hp

    @pl.when(i == 0)
    def _():
        acc_ref[...] = jnp.zeros_like(acc_ref)

    acc_ref[0:1, :] += jnp.sum(hp, axis=0, keepdims=True)
    acc_ref[1:2, :] += jnp.sum(hp * hp, axis=0, keepdims=True)

    @pl.when(i == nb - 1)
    def _():
        stats_ref[...] = acc_ref[...]


def _tc_b2_body(N, hpre_ref, stats_ref, g_ref, b_ref, hcat_ref):
    inv_n = 1.0 / N
    mu = stats_ref[0:1, :] * inv_n
    var = stats_ref[1:2, :] * inv_n - mu * mu
    scale = g_ref[...] * lax.rsqrt(var + 1e-5)
    shift = b_ref[...] - mu * scale
    h = hpre_ref[...] * scale + shift
    h = jnp.where(h >= 0, h, 0.01 * h)
    d = h.shape[1] // 2
    hcat_ref[0] = h[:, :d]
    hcat_ref[1] = h[:, d:]


def _tc_d_body(sum2_ref, rcp_ref, hcat_ref, wl_ref, bl_ref, wr_ref, out_ref):
    rcp = rcp_ref[:, 0:1]
    d = wl_ref.shape[0] // 2
    out_ref[...] = (
        jnp.dot(sum2_ref[0] * rcp, wl_ref[:d], preferred_element_type=_F32)
        + jnp.dot(sum2_ref[1] * rcp, wl_ref[d:], preferred_element_type=_F32)
        + jnp.dot(hcat_ref[0], wr_ref[:d], preferred_element_type=_F32)
        + jnp.dot(hcat_ref[1], wr_ref[d:], preferred_element_type=_F32)
        + bl_ref[...]
    )


def kernel(x, edge_index, W_l1, b_l1, W_r1, gamma1, beta1, W_l2, b_l2, W_r2):
    N, d_in = x.shape
    E = edge_index.shape[1]
    d_h = W_l1.shape[0]
    d_out = W_l2.shape[0]
    dh2 = d_h // 2
    NP = _round_up(N, 8 * NS)
    rz = NP // NS

    src = edge_index[0]
    dst = edge_index[1]
    np1 = E // NW // CH // PB
    np2 = E // NS // CH // PB
    src3 = src.reshape(NW, np1, PB, CH)
    dst3 = dst.reshape(NW, np1, PB, CH)
    src4 = jnp.concatenate([src, src + N]).reshape(NC, NS, np2, PB, CH)
    dst3b = dst.reshape(NS, np2, PB, CH)
    znf = jnp.zeros((rz, d_in), _F32)
    znh = jnp.zeros((rz, dh2), _F32)
    ones = jnp.ones((CH, d_in), _F32)

    sum1, cntp = _build_agg1(N, E, d_in)(x, src3, dst3, znf, ones)

    nb = 10
    R = N // nb
    hpre, stats, rcp16 = pl.pallas_call(
        functools.partial(_tc_b_body, nb),
        grid=(nb,),
        in_specs=[
            pl.BlockSpec((NC, R, d_in), lambda i: (0, i, 0)),
            pl.BlockSpec((NC, R, d_in), lambda i: (0, i, 0)),
            pl.BlockSpec((R, d_in), lambda i: (i, 0)),
            pl.BlockSpec((d_in, d_h), lambda i: (0, 0)),
            pl.BlockSpec((1, d_h), lambda i: (0, 0)),
            pl.BlockSpec((d_in, d_h), lambda i: (0, 0)),
        ],
        out_specs=[
            pl.BlockSpec((R, d_h), lambda i: (i, 0)),
            pl.BlockSpec((8, d_h), lambda i: (0, 0)),
            pl.BlockSpec((R, 16), lambda i: (i, 0)),
        ],
        out_shape=[
            jax.ShapeDtypeStruct((N, d_h), _F32),
            jax.ShapeDtypeStruct((8, d_h), _F32),
            jax.ShapeDtypeStruct((N, 16), _F32),
        ],
        scratch_shapes=[pltpu.VMEM((8, d_h), _F32)],
    )(sum1, cntp, x, W_l1.T, b_l1.reshape(1, -1), W_r1.T)

    hcat = pl.pallas_call(
        functools.partial(_tc_b2_body, N),
        grid=(nb,),
        in_specs=[
            pl.BlockSpec((R, d_h), lambda i: (i, 0)),
            pl.BlockSpec((8, d_h), lambda i: (0, 0)),
            pl.BlockSpec((1, d_h), lambda i: (0, 0)),
            pl.BlockSpec((1, d_h), lambda i: (0, 0)),
        ],
        out_specs=pl.BlockSpec((NC, R, dh2), lambda i: (0, i, 0)),
        out_shape=jax.ShapeDtypeStruct((NC, N, dh2), _F32),
    )(hpre, stats, gamma1.reshape(1, -1), beta1.reshape(1, -1))

    sum2 = _build_agg2(N, E, dh2)(hcat.reshape(NC * N, dh2), src4, dst3b, znh)

    node_emb = pl.pallas_call(
        _tc_d_body,
        grid=(nb,),
        in_specs=[
            pl.BlockSpec((NC, R, dh2), lambda i: (0, i, 0)),
            pl.BlockSpec((R, 16), lambda i: (i, 0)),
            pl.BlockSpec((NC, R, dh2), lambda i: (0, i, 0)),
            pl.BlockSpec((d_h, d_out), lambda i: (0, 0)),
            pl.BlockSpec((1, d_out), lambda i: (0, 0)),
            pl.BlockSpec((d_h, d_out), lambda i: (0, 0)),
        ],
        out_specs=pl.BlockSpec((R, d_out), lambda i: (i, 0)),
        out_shape=jax.ShapeDtypeStruct((N, d_out), _F32),
    )(sum2, rcp16, hcat, W_l2.T, b_l2.reshape(1, -1), W_r2.T)

    return node_emb


# speedup vs baseline: 9.5705x; 1.0048x over previous
import functools

import jax
import jax.numpy as jnp
from jax import lax
from jax.experimental import pallas as pl
from jax.experimental.pallas import tpu as pltpu
from jax.experimental.pallas import tpu_sc as plsc

NC = 2
NS = 16
NW = NC * NS
CH = 80

_F32 = jnp.float32


def _sc_mesh():
    return plsc.VectorSubcoreMesh(
        core_axis_name="c", subcore_axis_name="s", num_cores=NC, num_subcores=NS
    )


def _round_up(v, m):
    return (v + m - 1) // m * m


def _copy_out_slices(N, rz):
    slices = []
    for s in range(NS):
        r0 = s * rz
        slices.append((r0, min(rz, N - r0) if r0 < N else 0))
    return slices


def _pipelined_pass(table_hbm, acc, src_blk, dst_blk, rows, sem, sem2, pb):
    pltpu.async_copy(table_hbm.at[src_blk.at[0]], rows[0], sem)
    pltpu.async_copy(table_hbm.at[src_blk.at[1]], rows[1], sem)

    def step(i, carry):
        pltpu.make_async_copy(table_hbm.at[pl.ds(0, CH)], rows[0], sem).wait()

        @pl.when(i > 0)
        def _():
            pltpu.make_async_copy(rows[0], acc.at[dst_blk.at[0]], sem2).wait()

        @pl.when(i + 2 < pb)
        def _():
            for r in range(3):
                @pl.when(lax.rem(i, 3) == r)
                def _():
                    pltpu.async_copy(table_hbm.at[src_blk.at[i + 2]],
                                     rows[(r + 2) % 3], sem)

        for r in range(3):
            @pl.when(lax.rem(i, 3) == r)
            def _():
                pltpu.async_copy(rows[r], acc.at[dst_blk.at[i]], sem2,
                                 add=True)
        return carry

    lax.fori_loop(0, pb, step, 0)
    pltpu.make_async_copy(rows[0], acc.at[dst_blk.at[0]], sem2).wait()


PB = 25


@functools.cache
def _build_agg1(N, E, D):
    epw = E // NW
    nch = epw // CH
    NP = _round_up(N, 8 * NS)
    rz = NP // NS

    @functools.partial(
        pl.kernel,
        out_type=(
            jax.ShapeDtypeStruct((NC, N, D), _F32),
            jax.ShapeDtypeStruct((NC, N, D), _F32),
        ),
        mesh=_sc_mesh(),
        scratch_types=[
            pltpu.VMEM_SHARED((NP, D), _F32),
            pltpu.VMEM((PB, CH), jnp.int32),
            pltpu.VMEM((PB, CH), jnp.int32),
            pltpu.VMEM((CH, D), _F32),
            pltpu.VMEM((CH, D), _F32),
            pltpu.VMEM((CH, D), _F32),
            pltpu.SemaphoreType.DMA,
            pltpu.SemaphoreType.DMA,
        ],
    )
    def agg1(x_hbm, src3_hbm, dst3_hbm, znf_hbm, ones_hbm,
             sum_out, cnt_out, acc, src_blk, dst_blk, rows_a, rows_b, rows_c,
             sem, sem2):
        c = lax.axis_index("c")
        s = lax.axis_index("s")
        wid = s * NC + c
        z0 = pl.multiple_of(s * rz, 8)
        pltpu.sync_copy(znf_hbm, acc.at[pl.ds(z0, rz)])
        plsc.subcore_barrier()

        def ppass(p, carry):
            pltpu.sync_copy(src3_hbm.at[wid, p], src_blk)
            pltpu.sync_copy(dst3_hbm.at[wid, p], dst_blk)
            _pipelined_pass(x_hbm, acc, src_blk, dst_blk,
                            (rows_a, rows_b, rows_c), sem, sem2, PB)
            return carry

        lax.fori_loop(0, nch // PB, ppass, 0)
        plsc.subcore_barrier()
        for r0, nr in _copy_out_slices(N, rz):
            @pl.when(s == r0 // rz)
            def _():
                pltpu.sync_copy(acc.at[pl.ds(r0, nr)],
                                sum_out.at[c, pl.ds(r0, nr)])
        plsc.subcore_barrier()
        pltpu.sync_copy(znf_hbm, acc.at[pl.ds(z0, rz)])
        pltpu.sync_copy(ones_hbm, rows_a)
        plsc.subcore_barrier()

        def cpass(p, carry):
            pltpu.sync_copy(dst3_hbm.at[wid, p], dst_blk)

            def cfire(i, carry2):
                pltpu.async_copy(rows_a, acc.at[dst_blk.at[i]], sem2,
                                 add=True)
                return carry2

            lax.fori_loop(0, PB, cfire, 0)

            def cdrain(i, carry2):
                pltpu.make_async_copy(rows_a, acc.at[dst_blk.at[0]],
                                      sem2).wait()
                return carry2

            lax.fori_loop(0, PB, cdrain, 0)
            return carry

        lax.fori_loop(0, nch // PB, cpass, 0)
        plsc.subcore_barrier()
        for r0, nr in _copy_out_slices(N, rz):
            @pl.when(s == r0 // rz)
            def _():
                pltpu.sync_copy(acc.at[pl.ds(r0, nr)],
                                cnt_out.at[c, pl.ds(r0, nr)])

    return agg1


@functools.cache
def _build_agg2(N, E, D):
    ept = E // NS
    nch = ept // CH
    NP = _round_up(N, 8 * NS)
    rz = NP // NS

    @functools.partial(
        pl.kernel,
        out_type=jax.ShapeDtypeStruct((NC, N, D), _F32),
        mesh=_sc_mesh(),
        scratch_types=[
            pltpu.VMEM_SHARED((NP, D), _F32),
            pltpu.VMEM((PB, CH), jnp.int32),
            pltpu.VMEM((PB, CH), jnp.int32),
            pltpu.VMEM((CH, D), _F32),
            pltpu.VMEM((CH, D), _F32),
            pltpu.VMEM((CH, D), _F32),
            pltpu.SemaphoreType.DMA,
            pltpu.SemaphoreType.DMA,
        ],
    )
    def agg2(hcat_hbm, src4_hbm, dst3_hbm, znf_hbm,
             sum_out, acc, src_blk, dst_blk, rows_a, rows_b, rows_c,
             sem, sem2):
        c = lax.axis_index("c")
        s = lax.axis_index("s")
        z0 = pl.multiple_of(s * rz, 8)
        pltpu.sync_copy(znf_hbm, acc.at[pl.ds(z0, rz)])
        plsc.subcore_barrier()

        def ppass(p, carry):
            pltpu.sync_copy(src4_hbm.at[c, s, p], src_blk)
            pltpu.sync_copy(dst3_hbm.at[s, p], dst_blk)
            _pipelined_pass(hcat_hbm, acc, src_blk, dst_blk,
                            (rows_a, rows_b, rows_c), sem, sem2, PB)
            return carry

        lax.fori_loop(0, nch // PB, ppass, 0)
        plsc.subcore_barrier()
        for r0, nr in _copy_out_slices(N, rz):
            @pl.when(s == r0 // rz)
            def _():
                pltpu.sync_copy(acc.at[pl.ds(r0, nr)],
                                sum_out.at[c, pl.ds(r0, nr)])

    return agg2


def _tc_b_body(nb, N, R, sum1_ref, cntp_ref, x_ref, wl_ref, bl_ref, wr_ref,
               g_ref, b_ref, hcat_ref, hpre_scr, acc_ref):
    ph = pl.program_id(0)
    j = pl.program_id(1)

    @pl.when(ph == 0)
    def _():
        cnt = cntp_ref[0, :, 0:1] + cntp_ref[1, :, 0:1]
        rcp = 1.0 / jnp.clip(cnt, 1.0, None)
        mean1 = (sum1_ref[0] + sum1_ref[1]) * rcp
        hp = (jnp.dot(mean1, wl_ref[...], preferred_element_type=_F32)
              + jnp.dot(x_ref[...], wr_ref[...], preferred_element_type=_F32)
              + bl_ref[...])
        hpre_scr[pl.ds(j * R, R), :] = hp

        @pl.when(j == 0)
        def _():
            acc_ref[...] = jnp.zeros_like(acc_ref)

        acc_ref[0:1, :] += jnp.sum(hp, axis=0, keepdims=True)
        acc_ref[1:2, :] += jnp.sum(hp * hp, axis=0, keepdims=True)
        d = hp.shape[1] // 2
        hcat_ref[0] = hp[:, :d]
        hcat_ref[1] = hp[:, d:]

    @pl.when(ph == 1)
    def _():
        inv_n = 1.0 / N
        mu = acc_ref[0:1, :] * inv_n
        var = acc_ref[1:2, :] * inv_n - mu * mu
        scale = g_ref[...] * lax.rsqrt(var + 1e-5)
        shift = b_ref[...] - mu * scale
        h = hpre_scr[pl.ds(j * R, R), :] * scale + shift
        h = jnp.where(h >= 0, h, 0.01 * h)
        d = h.shape[1] // 2
        hcat_ref[0] = h[:, :d]
        hcat_ref[1] = h[:, d:]


def _tc_d_body(sum2_ref, cntp_ref, hcat_ref, wl_ref, bl_ref, wr_ref, out_ref):
    cnt = cntp_ref[0, :, 0:1] + cntp_ref[1, :, 0:1]
    rcp = 1.0 / jnp.clip(cnt, 1.0, None)
    d = wl_ref.shape[0] // 2
    out_ref[...] = (
        jnp.dot(sum2_ref[0] * rcp, wl_ref[:d], preferred_element_type=_F32)
        + jnp.dot(sum2_ref[1] * rcp, wl_ref[d:], preferred_element_type=_F32)
        + jnp.dot(hcat_ref[0], wr_ref[:d], preferred_element_type=_F32)
        + jnp.dot(hcat_ref[1], wr_ref[d:], preferred_element_type=_F32)
        + bl_ref[...]
    )


def kernel(x, edge_index, W_l1, b_l1, W_r1, gamma1, beta1, W_l2, b_l2, W_r2):
    N, d_in = x.shape
    E = edge_index.shape[1]
    d_h = W_l1.shape[0]
    d_out = W_l2.shape[0]
    dh2 = d_h // 2
    NP = _round_up(N, 8 * NS)
    rz = NP // NS

    src = edge_index[0]
    dst = edge_index[1]
    np1 = E // NW // CH // PB
    np2 = E // NS // CH // PB
    src3 = src.reshape(NW, np1, PB, CH)
    dst3 = dst.reshape(NW, np1, PB, CH)
    src4 = jnp.concatenate([src, src + N]).reshape(NC, NS, np2, PB, CH)
    dst3b = dst.reshape(NS, np2, PB, CH)
    znf = jnp.zeros((rz, d_in), _F32)
    znh = jnp.zeros((rz, dh2), _F32)
    ones = jnp.ones((CH, d_in), _F32)

    sum1, cntp = _build_agg1(N, E, d_in)(x, src3, dst3, znf, ones)

    nb = 10
    R = N // nb
    hcat = pl.pallas_call(
        functools.partial(_tc_b_body, nb, N, R),
        grid=(2, nb),
        in_specs=[
            pl.BlockSpec((NC, R, d_in), lambda p, j: (0, j * (1 - p), 0)),
            pl.BlockSpec((NC, R, d_in), lambda p, j: (0, j * (1 - p), 0)),
            pl.BlockSpec((R, d_in), lambda p, j: (j * (1 - p), 0)),
            pl.BlockSpec((d_in, d_h), lambda p, j: (0, 0)),
            pl.BlockSpec((1, d_h), lambda p, j: (0, 0)),
            pl.BlockSpec((d_in, d_h), lambda p, j: (0, 0)),
            pl.BlockSpec((1, d_h), lambda p, j: (0, 0)),
            pl.BlockSpec((1, d_h), lambda p, j: (0, 0)),
        ],
        out_specs=pl.BlockSpec((NC, R, dh2), lambda p, j: (0, j, 0)),
        out_shape=jax.ShapeDtypeStruct((NC, N, dh2), _F32),
        scratch_shapes=[
            pltpu.VMEM((N, d_h), _F32),
            pltpu.VMEM((8, d_h), _F32),
        ],
    )(sum1, cntp, x, W_l1.T, b_l1.reshape(1, -1), W_r1.T,
      gamma1.reshape(1, -1), beta1.reshape(1, -1))

    sum2 = _build_agg2(N, E, dh2)(hcat.reshape(NC * N, dh2), src4, dst3b, znh)

    node_emb = pl.pallas_call(
        _tc_d_body,
        grid=(nb,),
        in_specs=[
            pl.BlockSpec((NC, R, dh2), lambda i: (0, i, 0)),
            pl.BlockSpec((NC, R, d_in), lambda i: (0, i, 0)),
            pl.BlockSpec((NC, R, dh2), lambda i: (0, i, 0)),
            pl.BlockSpec((d_h, d_out), lambda i: (0, 0)),
            pl.BlockSpec((1, d_out), lambda i: (0, 0)),
            pl.BlockSpec((d_h, d_out), lambda i: (0, 0)),
        ],
        out_specs=pl.BlockSpec((R, d_out), lambda i: (i, 0)),
        out_shape=jax.ShapeDtypeStruct((N, d_out), _F32),
    )(sum2, cntp, hcat, W_l2.T, b_l2.reshape(1, -1), W_r2.T)

    return node_emb
